# baseline (speedup 1.0000x reference)
import jax
import jax.numpy as jnp
from jax import lax
from jax.experimental import pallas as pl
from jax.experimental.pallas import tpu as pltpu

_N = 471
_TOPK = 50
_ROWS = 4096
_FOLD = 8
_WIDE = _N * _FOLD
_WROWS = _ROWS // _FOLD
_BLK = 128
_NBLK = _WROWS // _BLK
_QR = 64
_NS_ITERS = 10
_ROUNDS = 10


def _body(x_ref, qt_ref, w_ref, linw_ref, linb_ref,
          alpha_ref, lamda_ref, rho_ref, mu_ref, out_ref, acc_ref, m_ref):
    i = pl.program_id(0)
    alpha = alpha_ref[0]
    lamda = lamda_ref[0]
    rho = rho_ref[0]
    mu = mu_ref[0]

    @pl.when(i == 0)
    def _setup():
        acc_ref[...] = jnp.zeros_like(acc_ref)
        qt = qt_ref[...]
        g = lax.dot_general(qt, qt, (((1,), (1,)), ((), ())),
                            preferred_element_type=jnp.float32)
        r = jnp.max(jnp.sum(jnp.abs(g), axis=1))
        eye = (jax.lax.broadcasted_iota(jnp.int32, (_QR, _QR), 0)
               == jax.lax.broadcasted_iota(jnp.int32, (_QR, _QR), 1))
        xinv = eye.astype(jnp.float32) * (1.0 / r)
        for _ in range(_NS_ITERS):
            gx = jnp.dot(g, xinv, preferred_element_type=jnp.float32)
            xinv = 2.0 * xinv - jnp.dot(xinv, gx,
                                        preferred_element_type=jnp.float32)
        t1 = jnp.dot(xinv, qt, preferred_element_type=jnp.float32)
        m_ref[...] = (alpha / _N) * lax.dot_general(
            qt, t1, (((0,), (0,)), ((), ())),
            preferred_element_type=jnp.float32)

    xb = x_ref[...]
    rows = _BLK
    while rows > 8:
        rows //= 2
        xb = xb[:rows, :] + xb[rows:2 * rows, :]
    acc_ref[...] += xb

    @pl.when(i == _NBLK - 1)
    def _admm():
        w = w_ref[...]
        m = m_ref[...]
        c = jnp.sum(acc_ref[...], axis=0, keepdims=True)
        s = c[:, 0:_N]
        for k in range(1, _FOLD):
            s = s + c[:, k * _N:(k + 1) * _N]
        a_mean = s * (1.0 / _ROWS)
        w2 = lax.dot_general(a_mean, linw_ref[...], (((1,), (1,)), ((), ())),
                             preferred_element_type=jnp.float32) + linb_ref[...]
        ones = jnp.ones_like(w)

        z = jnp.zeros_like(w)
        v = rho * w
        for _ in range(_ROUNDS):
            b = w + lax.dot_general(v, m, (((1,), (0,)), ((), ())),
                                    preferred_element_type=jnp.float32)
            grad = (w2 + v + rho * (2.0 * z - b - w)
                    + (2.0 * lamda) * (jnp.sum(z) - 1.0) * ones)
            z2 = jnp.maximum(z - mu * grad, 0.0)
            zc = z2.reshape(_N, 1)
            rank = jnp.sum((zc > z2).astype(jnp.float32), axis=0, keepdims=True)
            z = jnp.where(rank < float(_TOPK), z2, 0.0)
            v = v + rho * (z - b)
        out_ref[...] = z / (jnp.sum(z) + 1e-8)


def kernel(x, q_t, w, b1, alpha, lamda, rho, mu, lin_W, lin_b):
    del b1
    xw = x.reshape(_WROWS, _WIDE)
    w2d = w.reshape(1, _N)
    linb2d = lin_b.reshape(1, _N)
    smem = pl.BlockSpec(memory_space=pltpu.SMEM)
    out = pl.pallas_call(
        _body,
        grid=(_NBLK,),
        in_specs=[
            pl.BlockSpec((_BLK, _WIDE), lambda i: (i, 0)),
            pl.BlockSpec((_QR, _N), lambda i: (0, 0)),
            pl.BlockSpec((1, _N), lambda i: (0, 0)),
            pl.BlockSpec((_N, _N), lambda i: (0, 0)),
            pl.BlockSpec((1, _N), lambda i: (0, 0)),
            smem, smem, smem, smem,
        ],
        out_specs=pl.BlockSpec((1, _N), lambda i: (0, 0)),
        out_shape=jax.ShapeDtypeStruct((1, _N), jnp.float32),
        scratch_shapes=[pltpu.VMEM((8, _WIDE), jnp.float32),
                        pltpu.VMEM((_N, _N), jnp.float32)],
    )(xw, q_t, w2d, lin_W, linb2d, alpha, lamda, rho, mu)
    return out.reshape(_N)


# speedup vs baseline: 2.2016x; 2.2016x over previous
import jax
import jax.numpy as jnp
from jax import lax
from jax.experimental import pallas as pl
from jax.experimental.pallas import tpu as pltpu

_N = 471
_TOPK = 50
_ROWS = 4096
_NSTREAM = 4
_BLK = 512
_GRID = _ROWS // (_NSTREAM * _BLK)
_QR = 64
_NS_ITERS = 10
_ROUNDS = 10


def _tree8(xb):
    rows = xb.shape[0]
    while rows > 8:
        rows //= 2
        xb = xb[:rows, :] + xb[rows:2 * rows, :]
    return xb


def _body(x0_ref, x1_ref, x2_ref, x3_ref, qt_ref, w_ref, linw_ref, linb_ref,
          alpha_ref, lamda_ref, rho_ref, mu_ref, out_ref, acc_ref, m_ref):
    i = pl.program_id(0)
    alpha = alpha_ref[0]
    lamda = lamda_ref[0]
    rho = rho_ref[0]
    mu = mu_ref[0]

    @pl.when(i == 0)
    def _setup():
        qt = qt_ref[...]
        g = lax.dot_general(qt, qt, (((1,), (1,)), ((), ())),
                            preferred_element_type=jnp.float32)
        r = jnp.max(jnp.sum(jnp.abs(g), axis=1))
        eye = (jax.lax.broadcasted_iota(jnp.int32, (_QR, _QR), 0)
               == jax.lax.broadcasted_iota(jnp.int32, (_QR, _QR), 1))
        xinv = eye.astype(jnp.float32) * (1.0 / r)
        for _ in range(_NS_ITERS):
            gx = jnp.dot(g, xinv, preferred_element_type=jnp.float32)
            xinv = 2.0 * xinv - jnp.dot(xinv, gx,
                                        preferred_element_type=jnp.float32)
        t1 = jnp.dot(xinv, qt, preferred_element_type=jnp.float32)
        m_ref[...] = (alpha / _N) * lax.dot_general(
            qt, t1, (((0,), (0,)), ((), ())),
            preferred_element_type=jnp.float32)

    part = (_tree8(x0_ref[...]) + _tree8(x1_ref[...])
            + _tree8(x2_ref[...]) + _tree8(x3_ref[...]))

    @pl.when(i == 0)
    def _store0():
        acc_ref[...] = part

    @pl.when(i > 0)
    def _accum():
        acc_ref[...] += part

    @pl.when(i == _GRID - 1)
    def _admm():
        w = w_ref[...]
        m = m_ref[...]
        a_mean = jnp.sum(acc_ref[...], axis=0, keepdims=True) * (1.0 / _ROWS)
        w2 = lax.dot_general(a_mean, linw_ref[...], (((1,), (1,)), ((), ())),
                             preferred_element_type=jnp.float32) + linb_ref[...]
        ones = jnp.ones_like(w)

        z = jnp.zeros_like(w)
        v = rho * w
        for _ in range(_ROUNDS):
            b = w + lax.dot_general(v, m, (((1,), (0,)), ((), ())),
                                    preferred_element_type=jnp.float32)
            grad = (w2 + v + rho * (2.0 * z - b - w)
                    + (2.0 * lamda) * (jnp.sum(z) - 1.0) * ones)
            z2 = jnp.maximum(z - mu * grad, 0.0)
            zc = z2.reshape(_N, 1)
            rank = jnp.sum((zc > z2).astype(jnp.float32), axis=0, keepdims=True)
            z = jnp.where(rank < float(_TOPK), z2, 0.0)
            v = v + rho * (z - b)
        out_ref[...] = z / (jnp.sum(z) + 1e-8)


def kernel(x, q_t, w, b1, alpha, lamda, rho, mu, lin_W, lin_b):
    del b1
    w2d = w.reshape(1, _N)
    linb2d = lin_b.reshape(1, _N)
    smem = pl.BlockSpec(memory_space=pltpu.SMEM)
    xspecs = [
        pl.BlockSpec((_BLK, _N), lambda i, j=j: (_GRID * j + i, 0))
        for j in range(_NSTREAM)
    ]
    out = pl.pallas_call(
        _body,
        grid=(_GRID,),
        in_specs=xspecs + [
            pl.BlockSpec((_QR, _N), lambda i: (0, 0)),
            pl.BlockSpec((1, _N), lambda i: (0, 0)),
            pl.BlockSpec((_N, _N), lambda i: (0, 0)),
            pl.BlockSpec((1, _N), lambda i: (0, 0)),
            smem, smem, smem, smem,
        ],
        out_specs=pl.BlockSpec((1, _N), lambda i: (0, 0)),
        out_shape=jax.ShapeDtypeStruct((1, _N), jnp.float32),
        scratch_shapes=[pltpu.VMEM((8, _N), jnp.float32),
                        pltpu.VMEM((_N, _N), jnp.float32)],
    )(x, x, x, x, q_t, w2d, lin_W, linb2d, alpha, lamda, rho, mu)
    return out.reshape(_N)


# speedup vs baseline: 2.4004x; 1.0903x over previous
import jax
import jax.numpy as jnp
from jax import lax
from jax.experimental import pallas as pl
from jax.experimental.pallas import tpu as pltpu

_N = 471
_TOPK = 50
_ROWS = 4096
_NCHUNK = 8
_CHUNK = _ROWS // _NCHUNK
_QR = 64
_NS_ITERS = 10
_ROUNDS = 10


def _tree8(xb):
    rows = xb.shape[0]
    while rows > 8:
        rows //= 2
        xb = xb[:rows, :] + xb[rows:2 * rows, :]
    return xb


def _body(x_ref, qt_ref, w_ref, linw_ref, linb_ref,
          alpha_ref, lamda_ref, rho_ref, mu_ref, out_ref,
          xv_ref, lw_ref, sems, lw_sem):
    alpha = alpha_ref[0]
    lamda = lamda_ref[0]
    rho = rho_ref[0]
    mu = mu_ref[0]

    pltpu.make_async_copy(linw_ref, lw_ref, lw_sem).start()
    for c in range(_NCHUNK):
        pltpu.make_async_copy(
            x_ref.at[pl.ds(c * _CHUNK, _CHUNK), :],
            xv_ref.at[pl.ds(c * _CHUNK, _CHUNK), :],
            sems.at[c]).start()

    qt = qt_ref[...]
    g = lax.dot_general(qt, qt, (((1,), (1,)), ((), ())),
                        preferred_element_type=jnp.float32)
    r = jnp.max(jnp.sum(jnp.abs(g), axis=1))
    eye = (jax.lax.broadcasted_iota(jnp.int32, (_QR, _QR), 0)
           == jax.lax.broadcasted_iota(jnp.int32, (_QR, _QR), 1))
    xinv = eye.astype(jnp.float32) * (1.0 / r)
    for _ in range(_NS_ITERS):
        gx = jnp.dot(g, xinv, preferred_element_type=jnp.float32)
        xinv = 2.0 * xinv - jnp.dot(xinv, gx,
                                    preferred_element_type=jnp.float32)
    t1 = jnp.dot(xinv, qt, preferred_element_type=jnp.float32)
    m = (alpha / _N) * lax.dot_general(
        qt, t1, (((0,), (0,)), ((), ())),
        preferred_element_type=jnp.float32)

    acc = None
    for c in range(_NCHUNK):
        pltpu.make_async_copy(
            x_ref.at[pl.ds(c * _CHUNK, _CHUNK), :],
            xv_ref.at[pl.ds(c * _CHUNK, _CHUNK), :],
            sems.at[c]).wait()
        part = _tree8(xv_ref[pl.ds(c * _CHUNK, _CHUNK), :])
        acc = part if acc is None else acc + part
    a_mean = jnp.sum(acc, axis=0, keepdims=True) * (1.0 / _ROWS)

    pltpu.make_async_copy(linw_ref, lw_ref, lw_sem).wait()
    w = w_ref[...]
    w2 = lax.dot_general(a_mean, lw_ref[...], (((1,), (1,)), ((), ())),
                         preferred_element_type=jnp.float32) + linb_ref[...]
    ones = jnp.ones_like(w)

    z = jnp.zeros_like(w)
    v = rho * w
    for _ in range(_ROUNDS):
        b = w + lax.dot_general(v, m, (((1,), (0,)), ((), ())),
                                preferred_element_type=jnp.float32)
        grad = (w2 + v + rho * (2.0 * z - b - w)
                + (2.0 * lamda) * (jnp.sum(z) - 1.0) * ones)
        z2 = jnp.maximum(z - mu * grad, 0.0)
        zc = z2.reshape(_N, 1)
        rank = jnp.sum((zc > z2).astype(jnp.float32), axis=0, keepdims=True)
        z = jnp.where(rank < float(_TOPK), z2, 0.0)
        v = v + rho * (z - b)
    out_ref[...] = z / (jnp.sum(z) + 1e-8)


def kernel(x, q_t, w, b1, alpha, lamda, rho, mu, lin_W, lin_b):
    del b1
    w2d = w.reshape(1, _N)
    linb2d = lin_b.reshape(1, _N)
    smem = pl.BlockSpec(memory_space=pltpu.SMEM)
    anym = pl.BlockSpec(memory_space=pl.ANY)
    out = pl.pallas_call(
        _body,
        in_specs=[
            anym,
            pl.BlockSpec((_QR, _N), lambda: (0, 0)),
            pl.BlockSpec((1, _N), lambda: (0, 0)),
            anym,
            pl.BlockSpec((1, _N), lambda: (0, 0)),
            smem, smem, smem, smem,
        ],
        out_specs=pl.BlockSpec((1, _N), lambda: (0, 0)),
        out_shape=jax.ShapeDtypeStruct((1, _N), jnp.float32),
        scratch_shapes=[pltpu.VMEM((_ROWS, _N), jnp.float32),
                        pltpu.VMEM((_N, _N), jnp.float32),
                        pltpu.SemaphoreType.DMA((_NCHUNK,)),
                        pltpu.SemaphoreType.DMA],
    )(x, q_t, w2d, lin_W, linb2d, alpha, lamda, rho, mu)
    return out.reshape(_N)


# speedup vs baseline: 4.0293x; 1.6786x over previous
import jax
import jax.numpy as jnp
from jax import lax
from jax.experimental import pallas as pl
from jax.experimental.pallas import tpu as pltpu

_N = 471
_TOPK = 50
_ROWS = 4096
_CHUNKS = (64, 64, 64, 64, 64, 64, 64, 23)
_QR = 64
_NS_ITERS = 10
_ROUNDS = 10


def _body(xt_ref, qt_ref, w_ref, linw_ref, linb_ref,
          alpha_ref, lamda_ref, rho_ref, mu_ref, out_ref,
          xv_ref, lw_ref, sems, lw_sem):
    alpha = alpha_ref[0]
    lamda = lamda_ref[0]
    rho = rho_ref[0]
    mu = mu_ref[0]

    pltpu.make_async_copy(linw_ref, lw_ref, lw_sem).start()
    off = 0
    for c, rows in enumerate(_CHUNKS):
        pltpu.make_async_copy(
            xt_ref.at[pl.ds(off, rows), :],
            xv_ref.at[pl.ds(off, rows), :],
            sems.at[c]).start()
        off += rows

    qt = qt_ref[...]
    g = lax.dot_general(qt, qt, (((1,), (1,)), ((), ())),
                        preferred_element_type=jnp.float32)
    r = jnp.max(jnp.sum(jnp.abs(g), axis=1))
    eye = (jax.lax.broadcasted_iota(jnp.int32, (_QR, _QR), 0)
           == jax.lax.broadcasted_iota(jnp.int32, (_QR, _QR), 1))
    xinv = eye.astype(jnp.float32) * (1.0 / r)
    for _ in range(_NS_ITERS):
        gx = jnp.dot(g, xinv, preferred_element_type=jnp.float32)
        xinv = 2.0 * xinv - jnp.dot(xinv, gx,
                                    preferred_element_type=jnp.float32)
    t1 = jnp.dot(xinv, qt, preferred_element_type=jnp.float32)
    m = (alpha / _N) * lax.dot_general(
        qt, t1, (((0,), (0,)), ((), ())),
        preferred_element_type=jnp.float32)

    parts = []
    off = 0
    for c, rows in enumerate(_CHUNKS):
        pltpu.make_async_copy(
            xt_ref.at[pl.ds(off, rows), :],
            xv_ref.at[pl.ds(off, rows), :],
            sems.at[c]).wait()
        parts.append(jnp.sum(xv_ref[pl.ds(off, rows), :], axis=1,
                             keepdims=True))
        off += rows
    a_col = jnp.concatenate(parts, axis=0)
    a_mean = a_col.reshape(1, _N) * (1.0 / _ROWS)

    pltpu.make_async_copy(linw_ref, lw_ref, lw_sem).wait()
    w = w_ref[...]
    w2 = lax.dot_general(a_mean, lw_ref[...], (((1,), (1,)), ((), ())),
                         preferred_element_type=jnp.float32) + linb_ref[...]
    ones = jnp.ones_like(w)

    z = jnp.zeros_like(w)
    v = rho * w
    for _ in range(_ROUNDS):
        b = w + lax.dot_general(v, m, (((1,), (0,)), ((), ())),
                                preferred_element_type=jnp.float32)
        grad = (w2 + v + rho * (2.0 * z - b - w)
                + (2.0 * lamda) * (jnp.sum(z) - 1.0) * ones)
        z2 = jnp.maximum(z - mu * grad, 0.0)
        zc = z2.reshape(_N, 1)
        rank = jnp.sum((zc > z2).astype(jnp.float32), axis=0, keepdims=True)
        z = jnp.where(rank < float(_TOPK), z2, 0.0)
        v = v + rho * (z - b)
    out_ref[...] = z / (jnp.sum(z) + 1e-8)


def kernel(x, q_t, w, b1, alpha, lamda, rho, mu, lin_W, lin_b):
    del b1
    xt = x.T
    w2d = w.reshape(1, _N)
    linb2d = lin_b.reshape(1, _N)
    smem = pl.BlockSpec(memory_space=pltpu.SMEM)
    anym = pl.BlockSpec(memory_space=pl.ANY)
    out = pl.pallas_call(
        _body,
        in_specs=[
            anym,
            pl.BlockSpec((_QR, _N), lambda: (0, 0)),
            pl.BlockSpec((1, _N), lambda: (0, 0)),
            anym,
            pl.BlockSpec((1, _N), lambda: (0, 0)),
            smem, smem, smem, smem,
        ],
        out_specs=pl.BlockSpec((1, _N), lambda: (0, 0)),
        out_shape=jax.ShapeDtypeStruct((1, _N), jnp.float32),
        scratch_shapes=[pltpu.VMEM((_N, _ROWS), jnp.float32),
                        pltpu.VMEM((_N, _N), jnp.float32),
                        pltpu.SemaphoreType.DMA((len(_CHUNKS),)),
                        pltpu.SemaphoreType.DMA],
    )(xt, q_t, w2d, lin_W, linb2d, alpha, lamda, rho, mu)
    return out.reshape(_N)


# speedup vs baseline: 4.0691x; 1.0099x over previous
import jax
import jax.numpy as jnp
from jax import lax
from jax.experimental import pallas as pl
from jax.experimental.pallas import tpu as pltpu

_N = 471
_TOPK = 50
_ROWS = 4096
_CHUNKS = (64, 64, 64, 64, 64, 64, 64, 23)
_QR = 64
_NS_ITERS = 9
_ROUNDS = 10


def _body(xt_ref, qt_ref, w_ref, linw_ref, linb_ref,
          alpha_ref, lamda_ref, rho_ref, mu_ref, out_ref,
          xv_ref, lw_ref, sems, lw_sem):
    alpha = alpha_ref[0]
    lamda = lamda_ref[0]
    rho = rho_ref[0]
    mu = mu_ref[0]

    pltpu.make_async_copy(linw_ref, lw_ref, lw_sem).start()
    off = 0
    for c, rows in enumerate(_CHUNKS):
        pltpu.make_async_copy(
            xt_ref.at[pl.ds(off, rows), :],
            xv_ref.at[pl.ds(off, rows), :],
            sems.at[c]).start()
        off += rows

    qt = qt_ref[...]
    g = lax.dot_general(qt, qt, (((1,), (1,)), ((), ())),
                        preferred_element_type=jnp.float32)
    r = jnp.max(jnp.sum(jnp.abs(g), axis=1))
    eye = (jax.lax.broadcasted_iota(jnp.int32, (_QR, _QR), 0)
           == jax.lax.broadcasted_iota(jnp.int32, (_QR, _QR), 1))
    xinv = eye.astype(jnp.float32) * (1.0 / r)

    parts = []
    offs = []
    off = 0
    for rows in _CHUNKS:
        offs.append(off)
        off += rows

    def _chunk_sum(c):
        pltpu.make_async_copy(
            xt_ref.at[pl.ds(offs[c], _CHUNKS[c]), :],
            xv_ref.at[pl.ds(offs[c], _CHUNKS[c]), :],
            sems.at[c]).wait()
        parts.append(jnp.sum(xv_ref[pl.ds(offs[c], _CHUNKS[c]), :], axis=1,
                             keepdims=True))

    for k in range(_NS_ITERS):
        gx = jnp.dot(g, xinv, preferred_element_type=jnp.float32)
        xinv = 2.0 * xinv - jnp.dot(xinv, gx,
                                    preferred_element_type=jnp.float32)
        if k < len(_CHUNKS):
            _chunk_sum(k)
    for c in range(_NS_ITERS, len(_CHUNKS)):
        _chunk_sum(c)
    t1 = jnp.dot(xinv, qt, preferred_element_type=jnp.float32)
    m = (alpha / _N) * lax.dot_general(
        qt, t1, (((0,), (0,)), ((), ())),
        preferred_element_type=jnp.float32)

    a_col = jnp.concatenate(parts, axis=0)
    a_mean = a_col.reshape(1, _N) * (1.0 / _ROWS)

    pltpu.make_async_copy(linw_ref, lw_ref, lw_sem).wait()
    w = w_ref[...]
    w2 = lax.dot_general(a_mean, lw_ref[...], (((1,), (1,)), ((), ())),
                         preferred_element_type=jnp.float32) + linb_ref[...]
    ones = jnp.ones_like(w)

    z = jnp.zeros_like(w)
    v = rho * w
    for _ in range(_ROUNDS):
        b = w + lax.dot_general(v, m, (((1,), (0,)), ((), ())),
                                preferred_element_type=jnp.float32)
        grad = (w2 + v + rho * (2.0 * z - b - w)
                + (2.0 * lamda) * (jnp.sum(z) - 1.0) * ones)
        z2 = jnp.maximum(z - mu * grad, 0.0)
        zc = z2.reshape(_N, 1)
        rank = jnp.sum((zc > z2).astype(jnp.float32), axis=0, keepdims=True)
        z = jnp.where(rank < float(_TOPK), z2, 0.0)
        v = v + rho * (z - b)
    out_ref[...] = z / (jnp.sum(z) + 1e-8)


def kernel(x, q_t, w, b1, alpha, lamda, rho, mu, lin_W, lin_b):
    del b1
    xt = x.T
    w2d = w.reshape(1, _N)
    linb2d = lin_b.reshape(1, _N)
    smem = pl.BlockSpec(memory_space=pltpu.SMEM)
    anym = pl.BlockSpec(memory_space=pl.ANY)
    out = pl.pallas_call(
        _body,
        in_specs=[
            anym,
            pl.BlockSpec((_QR, _N), lambda: (0, 0)),
            pl.BlockSpec((1, _N), lambda: (0, 0)),
            anym,
            pl.BlockSpec((1, _N), lambda: (0, 0)),
            smem, smem, smem, smem,
        ],
        out_specs=pl.BlockSpec((1, _N), lambda: (0, 0)),
        out_shape=jax.ShapeDtypeStruct((1, _N), jnp.float32),
        scratch_shapes=[pltpu.VMEM((_N, _ROWS), jnp.float32),
                        pltpu.VMEM((_N, _N), jnp.float32),
                        pltpu.SemaphoreType.DMA((len(_CHUNKS),)),
                        pltpu.SemaphoreType.DMA],
    )(xt, q_t, w2d, lin_W, linb2d, alpha, lamda, rho, mu)
    return out.reshape(_N)
